# Initial kernel scaffold; baseline (speedup 1.0000x reference)
import jax
import jax.numpy as jnp
from jax.experimental import pallas as pl


def kernel(x, pos, edge_index, edge_attr, node_attr, batch, W_emb, b_emb, W_msg1_0, b_msg1_0, W_msg2_0, b_msg2_0, W_upd1_0, b_upd1_0, W_upd2_0, b_upd2_0, W_msg1_1, b_msg1_1, W_msg2_1, b_msg2_1, W_upd1_1, b_upd1_1, W_upd2_1, b_upd2_1, W_pre1, b_pre1, W_pre2, b_pre2):
    raise NotImplementedError("write your pallas kernel here")


# speedup vs baseline: 1.2916x; 1.2916x over previous
import functools

import jax
import jax.numpy as jnp
import numpy as np
from jax.experimental import pallas as pl
from jax.experimental.pallas import tpu as pltpu

N = 10000
E = 160000
D = 128
A = 4

NODE_BLK = 2000
EDGE_BLK = 2000


def _stack_w(W):
    scale = 1.0 / np.sqrt(W.shape[1] * W.shape[2])
    return jnp.transpose(W, (2, 1, 0)) * scale


def _silu(v):
    return v * jax.nn.sigmoid(v)


def _tp_sum(x, attr, W_ref):
    acc = None
    for a in range(A):
        d = jnp.dot(x, W_ref[a], preferred_element_type=jnp.float32)
        d = d * attr[:, a : a + 1]
        acc = d if acc is None else acc + d
    return acc


def _two_stage_body(two_inputs, silu_last, residual):
    def body(*refs):
        if two_inputs:
            x1_ref, x2_ref, attr_ref, Wa_ref, ba_ref, Wb_ref, bb_ref, out_ref = refs
            x = jnp.concatenate([x1_ref[...], x2_ref[...]], axis=-1)
        else:
            x1_ref, attr_ref, Wa_ref, ba_ref, Wb_ref, bb_ref, out_ref = refs
            x = x1_ref[...]
        attr = attr_ref[...]
        h1 = _silu(_tp_sum(x, attr, Wa_ref) + ba_ref[...])
        o = _tp_sum(h1, attr, Wb_ref) + bb_ref[...]
        if silu_last:
            o = _silu(o)
        if residual:
            o = o + x1_ref[...]
        out_ref[...] = o

    return body


def _emb_body(x_ref, attr_ref, W_ref, b_ref, out_ref):
    out_ref[...] = _tp_sum(x_ref[...], attr_ref[...], W_ref) + b_ref[...]


def _full_spec(shape):
    nd = len(shape)
    return pl.BlockSpec(shape, lambda i, _n=nd: (0,) * _n)


def _tp2_call(x1, x2, attr, Wa, ba, Wb, bb, *, silu_last, residual, blk):
    M = x1.shape[0]
    assert M % blk == 0
    two = x2 is not None
    Was = _stack_w(Wa)
    Wbs = _stack_w(Wb)
    ba2 = ba.reshape(1, D)
    bb2 = bb.reshape(1, D)
    args = [x1] + ([x2] if two else []) + [attr, Was, ba2, Wbs, bb2]
    in_specs = [pl.BlockSpec((blk, D), lambda i: (i, 0))]
    if two:
        in_specs.append(pl.BlockSpec((blk, D), lambda i: (i, 0)))
    in_specs += [
        pl.BlockSpec((blk, A), lambda i: (i, 0)),
        _full_spec(Was.shape),
        _full_spec((1, D)),
        _full_spec(Wbs.shape),
        _full_spec((1, D)),
    ]
    return pl.pallas_call(
        _two_stage_body(two, silu_last, residual),
        grid=(M // blk,),
        in_specs=in_specs,
        out_specs=pl.BlockSpec((blk, D), lambda i: (i, 0)),
        out_shape=jax.ShapeDtypeStruct((M, D), jnp.float32),
    )(*args)


def _emb_call(x, attr, W, b, *, blk):
    M = x.shape[0]
    Ws = _stack_w(W)
    b2 = b.reshape(1, D)
    return pl.pallas_call(
        _emb_body,
        grid=(M // blk,),
        in_specs=[
            pl.BlockSpec((blk, D), lambda i: (i, 0)),
            pl.BlockSpec((blk, A), lambda i: (i, 0)),
            _full_spec(Ws.shape),
            _full_spec((1, D)),
        ],
        out_specs=pl.BlockSpec((blk, D), lambda i: (i, 0)),
        out_shape=jax.ShapeDtypeStruct((M, D), jnp.float32),
    )(x, attr, Ws, b2)


def kernel(x, pos, edge_index, edge_attr, node_attr, batch, W_emb, b_emb,
           W_msg1_0, b_msg1_0, W_msg2_0, b_msg2_0, W_upd1_0, b_upd1_0,
           W_upd2_0, b_upd2_0, W_msg1_1, b_msg1_1, W_msg2_1, b_msg2_1,
           W_upd1_1, b_upd1_1, W_upd2_1, b_upd2_1, W_pre1, b_pre1,
           W_pre2, b_pre2):
    na = node_attr.at[:, 0].set(1.0)
    h = _emb_call(x, na, W_emb, b_emb, blk=NODE_BLK)
    src = edge_index[0]
    dst = edge_index[1]
    layers = [
        (W_msg1_0, b_msg1_0, W_msg2_0, b_msg2_0, W_upd1_0, b_upd1_0, W_upd2_0, b_upd2_0),
        (W_msg1_1, b_msg1_1, W_msg2_1, b_msg2_1, W_upd1_1, b_upd1_1, W_upd2_1, b_upd2_1),
    ]
    for (Wm1, bm1, Wm2, bm2, Wu1, bu1, Wu2, bu2) in layers:
        hd = jnp.take(h, dst, axis=0)
        hs = jnp.take(h, src, axis=0)
        m2 = _tp2_call(hd, hs, edge_attr, Wm1, bm1, Wm2, bm2,
                       silu_last=True, residual=False, blk=EDGE_BLK)
        agg = jax.ops.segment_sum(m2, dst, num_segments=N)
        h = _tp2_call(h, agg, na, Wu1, bu1, Wu2, bu2,
                      silu_last=False, residual=True, blk=NODE_BLK)
    h = _tp2_call(h, None, na, W_pre1, b_pre1, W_pre2, b_pre2,
                  silu_last=False, residual=False, blk=NODE_BLK)
    return h


# speedup vs baseline: 3.6072x; 2.7928x over previous
import functools

import jax
import jax.numpy as jnp
import numpy as np
from jax import lax
from jax.experimental import pallas as pl
from jax.experimental.pallas import tpu as pltpu
from jax.experimental.pallas import tpu_sc as plsc

N = 10000
E = 160000
D = 128
A = 4

NODE_BLK = 2000
EDGE_BLK = 2000

SC_CORES = 2
SC_SUBCORES = 16
NW = SC_CORES * SC_SUBCORES
CH = 128
NCHUNK = E // CH
BASE_CH = NCHUNK // NW
EXTRA_CH = NCHUNK % NW
ROWS_PER_TILE = (N // SC_SUBCORES) // 8 * 8
ROWS_REMAINDER = N - SC_SUBCORES * ROWS_PER_TILE

_sc_mesh = plsc.VectorSubcoreMesh(core_axis_name="c", subcore_axis_name="s")


@functools.partial(
    pl.kernel,
    out_type=(jax.ShapeDtypeStruct((E, D), jnp.float32),
              jax.ShapeDtypeStruct((E, D), jnp.float32)),
    mesh=_sc_mesh,
    scratch_types=[
        pltpu.VMEM((CH,), jnp.int32),
        pltpu.VMEM((CH,), jnp.int32),
        pltpu.VMEM((CH, D), jnp.float32),
        pltpu.VMEM((CH, D), jnp.float32),
        pltpu.SemaphoreType.DMA,
        pltpu.SemaphoreType.DMA,
    ],
)
def _sc_gather2(h_hbm, dst_hbm, src_hbm, od_hbm, os_hbm,
                idxd_v, idxs_v, rowd_v, rows_v, semd, sems):
    wid = lax.axis_index("s") * SC_CORES + lax.axis_index("c")
    nch = BASE_CH + (wid < EXTRA_CH).astype(jnp.int32)

    def body(j, carry):
        base = (wid + j * NW) * CH
        pltpu.sync_copy(dst_hbm.at[pl.ds(base, CH)], idxd_v)
        pltpu.sync_copy(src_hbm.at[pl.ds(base, CH)], idxs_v)
        cpd = pltpu.async_copy(h_hbm.at[idxd_v], rowd_v, semd)
        cps = pltpu.async_copy(h_hbm.at[idxs_v], rows_v, sems)
        cpd.wait()
        cps.wait()
        pltpu.sync_copy(rowd_v, od_hbm.at[pl.ds(base, CH)])
        pltpu.sync_copy(rows_v, os_hbm.at[pl.ds(base, CH)])
        return carry

    lax.fori_loop(0, nch, body, 0)


@functools.partial(
    pl.kernel,
    out_type=jax.ShapeDtypeStruct((SC_CORES * N, D), jnp.float32),
    mesh=_sc_mesh,
    scratch_types=[
        pltpu.VMEM((CH,), jnp.int32),
        pltpu.VMEM((CH, D), jnp.float32),
        pltpu.VMEM_SHARED((N, D), jnp.float32),
        pltpu.SemaphoreType.DMA,
    ],
)
def _sc_scatter_add(m_hbm, dst_hbm, zeros_hbm, out_hbm,
                    idx_v, rows_v, agg_sh, sem):
    c = lax.axis_index("c")
    s = lax.axis_index("s")
    wid = s * SC_CORES + c
    nch = BASE_CH + (wid < EXTRA_CH).astype(jnp.int32)

    row0 = s * ROWS_PER_TILE
    pltpu.sync_copy(zeros_hbm.at[pl.ds(row0, ROWS_PER_TILE)],
                    agg_sh.at[pl.ds(row0, ROWS_PER_TILE)])

    @pl.when(s == SC_SUBCORES - 1)
    def _init_tail():
        tail0 = SC_SUBCORES * ROWS_PER_TILE
        pltpu.sync_copy(zeros_hbm.at[pl.ds(tail0, ROWS_REMAINDER)],
                        agg_sh.at[pl.ds(tail0, ROWS_REMAINDER)])

    plsc.subcore_barrier()

    def body(j, carry):
        base = (wid + j * NW) * CH
        pltpu.sync_copy(dst_hbm.at[pl.ds(base, CH)], idx_v)
        pltpu.async_copy(m_hbm.at[pl.ds(base, CH)], rows_v, sem).wait()
        pltpu.sync_copy(rows_v, agg_sh.at[idx_v], add=True)
        return carry

    lax.fori_loop(0, nch, body, 0)
    plsc.subcore_barrier()
    pltpu.sync_copy(agg_sh.at[pl.ds(row0, ROWS_PER_TILE)],
                    out_hbm.at[pl.ds(c * N + row0, ROWS_PER_TILE)])

    @pl.when(s == SC_SUBCORES - 1)
    def _out_tail():
        tail0 = SC_SUBCORES * ROWS_PER_TILE
        pltpu.sync_copy(agg_sh.at[pl.ds(tail0, ROWS_REMAINDER)],
                        out_hbm.at[pl.ds(c * N + tail0, ROWS_REMAINDER)])


def _stack_w(W):
    scale = 1.0 / np.sqrt(W.shape[1] * W.shape[2])
    return jnp.transpose(W, (2, 1, 0)) * scale


def _silu(v):
    return v * jax.nn.sigmoid(v)


def _tp_sum(x, attr, W_ref):
    acc = None
    for a in range(A):
        d = jnp.dot(x, W_ref[a], preferred_element_type=jnp.float32)
        d = d * attr[:, a : a + 1]
        acc = d if acc is None else acc + d
    return acc


def _two_stage_body(n_x2, silu_last, residual):
    def body(*refs):
        if n_x2 == 2:
            x1_ref, x2a_ref, x2b_ref, attr_ref, Wa_ref, ba_ref, Wb_ref, bb_ref, out_ref = refs
            x = jnp.concatenate([x1_ref[...], x2a_ref[...] + x2b_ref[...]], axis=-1)
        elif n_x2 == 1:
            x1_ref, x2_ref, attr_ref, Wa_ref, ba_ref, Wb_ref, bb_ref, out_ref = refs
            x = jnp.concatenate([x1_ref[...], x2_ref[...]], axis=-1)
        else:
            x1_ref, attr_ref, Wa_ref, ba_ref, Wb_ref, bb_ref, out_ref = refs
            x = x1_ref[...]
        attr = attr_ref[...]
        h1 = _silu(_tp_sum(x, attr, Wa_ref) + ba_ref[...])
        o = _tp_sum(h1, attr, Wb_ref) + bb_ref[...]
        if silu_last:
            o = _silu(o)
        if residual:
            o = o + x1_ref[...]
        out_ref[...] = o

    return body


def _emb_body(x_ref, attr_ref, W_ref, b_ref, out_ref):
    out_ref[...] = _tp_sum(x_ref[...], attr_ref[...], W_ref) + b_ref[...]


def _full_spec(shape):
    nd = len(shape)
    return pl.BlockSpec(shape, lambda i, _n=nd: (0,) * _n)


def _tp2_call(x1, x2s, attr, Wa, ba, Wb, bb, *, silu_last, residual, blk):
    M = x1.shape[0]
    assert M % blk == 0
    n_x2 = len(x2s)
    Was = _stack_w(Wa)
    Wbs = _stack_w(Wb)
    ba2 = ba.reshape(1, D)
    bb2 = bb.reshape(1, D)
    args = [x1] + list(x2s) + [attr, Was, ba2, Wbs, bb2]
    in_specs = [pl.BlockSpec((blk, D), lambda i: (i, 0))
                for _ in range(1 + n_x2)]
    in_specs += [
        pl.BlockSpec((blk, A), lambda i: (i, 0)),
        _full_spec(Was.shape),
        _full_spec((1, D)),
        _full_spec(Wbs.shape),
        _full_spec((1, D)),
    ]
    return pl.pallas_call(
        _two_stage_body(n_x2, silu_last, residual),
        grid=(M // blk,),
        in_specs=in_specs,
        out_specs=pl.BlockSpec((blk, D), lambda i: (i, 0)),
        out_shape=jax.ShapeDtypeStruct((M, D), jnp.float32),
    )(*args)


def _emb_call(x, attr, W, b, *, blk):
    M = x.shape[0]
    Ws = _stack_w(W)
    b2 = b.reshape(1, D)
    return pl.pallas_call(
        _emb_body,
        grid=(M // blk,),
        in_specs=[
            pl.BlockSpec((blk, D), lambda i: (i, 0)),
            pl.BlockSpec((blk, A), lambda i: (i, 0)),
            _full_spec(Ws.shape),
            _full_spec((1, D)),
        ],
        out_specs=pl.BlockSpec((blk, D), lambda i: (i, 0)),
        out_shape=jax.ShapeDtypeStruct((M, D), jnp.float32),
    )(x, attr, Ws, b2)


def kernel(x, pos, edge_index, edge_attr, node_attr, batch, W_emb, b_emb,
           W_msg1_0, b_msg1_0, W_msg2_0, b_msg2_0, W_upd1_0, b_upd1_0,
           W_upd2_0, b_upd2_0, W_msg1_1, b_msg1_1, W_msg2_1, b_msg2_1,
           W_upd1_1, b_upd1_1, W_upd2_1, b_upd2_1, W_pre1, b_pre1,
           W_pre2, b_pre2):
    na = node_attr.at[:, 0].set(1.0)
    h = _emb_call(x, na, W_emb, b_emb, blk=NODE_BLK)
    src = edge_index[0]
    dst = edge_index[1]
    zeros_nd = jnp.zeros((N, D), jnp.float32)
    layers = [
        (W_msg1_0, b_msg1_0, W_msg2_0, b_msg2_0, W_upd1_0, b_upd1_0, W_upd2_0, b_upd2_0),
        (W_msg1_1, b_msg1_1, W_msg2_1, b_msg2_1, W_upd1_1, b_upd1_1, W_upd2_1, b_upd2_1),
    ]
    for (Wm1, bm1, Wm2, bm2, Wu1, bu1, Wu2, bu2) in layers:
        hd, hs = _sc_gather2(h, dst, src)
        m2 = _tp2_call(hd, [hs], edge_attr, Wm1, bm1, Wm2, bm2,
                       silu_last=True, residual=False, blk=EDGE_BLK)
        agg2 = _sc_scatter_add(m2, dst, zeros_nd)
        h = _tp2_call(h, [agg2[:N], agg2[N:]], na, Wu1, bu1, Wu2, bu2,
                      silu_last=False, residual=True, blk=NODE_BLK)
    h = _tp2_call(h, [], na, W_pre1, b_pre1, W_pre2, b_pre2,
                  silu_last=False, residual=False, blk=NODE_BLK)
    return h


# speedup vs baseline: 4.4686x; 1.2388x over previous
import functools

import jax
import jax.numpy as jnp
import numpy as np
from jax import lax
from jax.experimental import pallas as pl
from jax.experimental.pallas import tpu as pltpu
from jax.experimental.pallas import tpu_sc as plsc

N = 10000
E = 160000
D = 128
A = 4

NODE_BLK = 2000
EDGE_BLK = 2000

SC_CORES = 2
SC_SUBCORES = 16
NW = SC_CORES * SC_SUBCORES
CH = 128
NCHUNK = E // CH
BASE_CH = NCHUNK // NW
EXTRA_CH = NCHUNK % NW
ROWS_PER_TILE = (N // SC_SUBCORES) // 8 * 8
ROWS_REMAINDER = N - SC_SUBCORES * ROWS_PER_TILE

_sc_mesh = plsc.VectorSubcoreMesh(core_axis_name="c", subcore_axis_name="s")


IDXBUF = (BASE_CH + 1) * CH
NGROUP = (BASE_CH + 2) // 2


@functools.partial(
    pl.kernel,
    out_type=(jax.ShapeDtypeStruct((E, D), jnp.float32),
              jax.ShapeDtypeStruct((E, D), jnp.float32)),
    mesh=_sc_mesh,
    scratch_types=[
        pltpu.VMEM((IDXBUF,), jnp.int32),
        pltpu.VMEM((IDXBUF,), jnp.int32),
        pltpu.VMEM((2, CH, D), jnp.float32),
        pltpu.VMEM((2, CH, D), jnp.float32),
        pltpu.SemaphoreType.DMA,
        pltpu.SemaphoreType.DMA,
        pltpu.SemaphoreType.DMA,
        pltpu.SemaphoreType.DMA,
    ],
)
def _sc_gather2(h_hbm, dst_hbm, src_hbm, od_hbm, os_hbm,
                idxd_all, idxs_all, rowd, rows, sd0, sd1, ss0, ss1):
    wid = lax.axis_index("s") * SC_CORES + lax.axis_index("c")
    nch = BASE_CH + (wid < EXTRA_CH).astype(jnp.int32)
    start = wid * BASE_CH + jnp.minimum(wid, EXTRA_CH)
    base0 = start * CH
    pltpu.sync_copy(dst_hbm.at[pl.ds(base0, IDXBUF)], idxd_all)
    pltpu.sync_copy(src_hbm.at[pl.ds(base0, IDXBUF)], idxs_all)
    semd = (sd0, sd1)
    sems = (ss0, ss1)

    def fire(k, b):
        pltpu.async_copy(h_hbm.at[idxd_all.at[pl.ds(k * CH, CH)]],
                         rowd.at[b], semd[b])
        pltpu.async_copy(h_hbm.at[idxs_all.at[pl.ds(k * CH, CH)]],
                         rows.at[b], sems[b])

    fire(0, 0)
    fire(1, 1)

    def group(g, carry):
        for b in range(2):
            k = 2 * g + b

            @pl.when(k < nch)
            def _drain():
                pltpu.make_async_copy(od_hbm.at[pl.ds(0, CH)],
                                      rowd.at[b], semd[b]).wait()
                pltpu.make_async_copy(od_hbm.at[pl.ds(0, CH)],
                                      rows.at[b], sems[b]).wait()
                pltpu.sync_copy(rowd.at[b], od_hbm.at[pl.ds(base0 + k * CH, CH)])
                pltpu.sync_copy(rows.at[b], os_hbm.at[pl.ds(base0 + k * CH, CH)])

                @pl.when(k + 2 < nch)
                def _refill():
                    fire(k + 2, b)

        return carry

    lax.fori_loop(0, NGROUP, group, 0)


@functools.partial(
    pl.kernel,
    out_type=jax.ShapeDtypeStruct((SC_CORES * N, D), jnp.float32),
    mesh=_sc_mesh,
    scratch_types=[
        pltpu.VMEM((CH,), jnp.int32),
        pltpu.VMEM((CH,), jnp.int32),
        pltpu.VMEM((2, CH, D), jnp.float32),
        pltpu.VMEM_SHARED((N, D), jnp.float32),
        pltpu.SemaphoreType.DMA,
        pltpu.SemaphoreType.DMA,
    ],
)
def _sc_scatter_add(m_hbm, dst_hbm, zeros_hbm, out_hbm,
                    idx0, idx1, rows, agg_sh, sm0, sm1):
    c = lax.axis_index("c")
    s = lax.axis_index("s")
    wid = s * SC_CORES + c
    nch = BASE_CH + (wid < EXTRA_CH).astype(jnp.int32)
    start = wid * BASE_CH + jnp.minimum(wid, EXTRA_CH)
    base0 = start * CH

    row0 = s * ROWS_PER_TILE
    pltpu.sync_copy(zeros_hbm.at[pl.ds(row0, ROWS_PER_TILE)],
                    agg_sh.at[pl.ds(row0, ROWS_PER_TILE)])

    @pl.when(s == SC_SUBCORES - 1)
    def _init_tail():
        tail0 = SC_SUBCORES * ROWS_PER_TILE
        pltpu.sync_copy(zeros_hbm.at[pl.ds(tail0, ROWS_REMAINDER)],
                        agg_sh.at[pl.ds(tail0, ROWS_REMAINDER)])

    plsc.subcore_barrier()

    sems = (sm0, sm1)
    idxs = (idx0, idx1)

    def fire(k, b):
        pltpu.async_copy(m_hbm.at[pl.ds(base0 + k * CH, CH)], rows.at[b], sems[b])
        pltpu.sync_copy(dst_hbm.at[pl.ds(base0 + k * CH, CH)], idxs[b])

    fire(0, 0)
    fire(1, 1)

    def group(g, carry):
        for b in range(2):
            k = 2 * g + b

            @pl.when(k < nch)
            def _drain():
                pltpu.make_async_copy(m_hbm.at[pl.ds(0, CH)],
                                      rows.at[b], sems[b]).wait()
                pltpu.sync_copy(rows.at[b], agg_sh.at[idxs[b]], add=True)

                @pl.when(k + 2 < nch)
                def _refill():
                    fire(k + 2, b)

        return carry

    lax.fori_loop(0, NGROUP, group, 0)
    plsc.subcore_barrier()
    pltpu.sync_copy(agg_sh.at[pl.ds(row0, ROWS_PER_TILE)],
                    out_hbm.at[pl.ds(c * N + row0, ROWS_PER_TILE)])

    @pl.when(s == SC_SUBCORES - 1)
    def _out_tail():
        tail0 = SC_SUBCORES * ROWS_PER_TILE
        pltpu.sync_copy(agg_sh.at[pl.ds(tail0, ROWS_REMAINDER)],
                        out_hbm.at[pl.ds(c * N + tail0, ROWS_REMAINDER)])


def _stack_w(W):
    scale = 1.0 / np.sqrt(W.shape[1] * W.shape[2])
    return jnp.transpose(W, (2, 1, 0)) * scale


def _silu(v):
    return v * jax.nn.sigmoid(v)


def _tp_sum(x, attr, W_ref):
    acc = None
    for a in range(A):
        d = jnp.dot(x, W_ref[a], preferred_element_type=jnp.float32)
        d = d * attr[:, a : a + 1]
        acc = d if acc is None else acc + d
    return acc


def _two_stage_body(n_x2, silu_last, residual):
    def body(*refs):
        if n_x2 == 2:
            x1_ref, x2a_ref, x2b_ref, attr_ref, Wa_ref, ba_ref, Wb_ref, bb_ref, out_ref = refs
            x = jnp.concatenate([x1_ref[...], x2a_ref[...] + x2b_ref[...]], axis=-1)
        elif n_x2 == 1:
            x1_ref, x2_ref, attr_ref, Wa_ref, ba_ref, Wb_ref, bb_ref, out_ref = refs
            x = jnp.concatenate([x1_ref[...], x2_ref[...]], axis=-1)
        else:
            x1_ref, attr_ref, Wa_ref, ba_ref, Wb_ref, bb_ref, out_ref = refs
            x = x1_ref[...]
        attr = attr_ref[...]
        h1 = _silu(_tp_sum(x, attr, Wa_ref) + ba_ref[...])
        o = _tp_sum(h1, attr, Wb_ref) + bb_ref[...]
        if silu_last:
            o = _silu(o)
        if residual:
            o = o + x1_ref[...]
        out_ref[...] = o

    return body


def _emb_body(x_ref, attr_ref, W_ref, b_ref, out_ref):
    out_ref[...] = _tp_sum(x_ref[...], attr_ref[...], W_ref) + b_ref[...]


def _full_spec(shape):
    nd = len(shape)
    return pl.BlockSpec(shape, lambda i, _n=nd: (0,) * _n)


def _tp2_call(x1, x2s, attr, Wa, ba, Wb, bb, *, silu_last, residual, blk):
    M = x1.shape[0]
    assert M % blk == 0
    n_x2 = len(x2s)
    Was = _stack_w(Wa)
    Wbs = _stack_w(Wb)
    ba2 = ba.reshape(1, D)
    bb2 = bb.reshape(1, D)
    args = [x1] + [a for (a, _) in x2s] + [attr, Was, ba2, Wbs, bb2]
    in_specs = [pl.BlockSpec((blk, D), lambda i: (i, 0))]
    in_specs += [pl.BlockSpec((blk, D), lambda i, _o=off: (i + _o, 0))
                 for (_, off) in x2s]
    in_specs += [
        pl.BlockSpec((blk, A), lambda i: (i, 0)),
        _full_spec(Was.shape),
        _full_spec((1, D)),
        _full_spec(Wbs.shape),
        _full_spec((1, D)),
    ]
    return pl.pallas_call(
        _two_stage_body(n_x2, silu_last, residual),
        grid=(M // blk,),
        in_specs=in_specs,
        out_specs=pl.BlockSpec((blk, D), lambda i: (i, 0)),
        out_shape=jax.ShapeDtypeStruct((M, D), jnp.float32),
    )(*args)


def _emb_call(x, attr, W, b, *, blk):
    M = x.shape[0]
    Ws = _stack_w(W)
    b2 = b.reshape(1, D)
    return pl.pallas_call(
        _emb_body,
        grid=(M // blk,),
        in_specs=[
            pl.BlockSpec((blk, D), lambda i: (i, 0)),
            pl.BlockSpec((blk, A), lambda i: (i, 0)),
            _full_spec(Ws.shape),
            _full_spec((1, D)),
        ],
        out_specs=pl.BlockSpec((blk, D), lambda i: (i, 0)),
        out_shape=jax.ShapeDtypeStruct((M, D), jnp.float32),
    )(x, attr, Ws, b2)


def kernel(x, pos, edge_index, edge_attr, node_attr, batch, W_emb, b_emb,
           W_msg1_0, b_msg1_0, W_msg2_0, b_msg2_0, W_upd1_0, b_upd1_0,
           W_upd2_0, b_upd2_0, W_msg1_1, b_msg1_1, W_msg2_1, b_msg2_1,
           W_upd1_1, b_upd1_1, W_upd2_1, b_upd2_1, W_pre1, b_pre1,
           W_pre2, b_pre2):
    na = node_attr.at[:, 0].set(1.0)
    h = _emb_call(x, na, W_emb, b_emb, blk=NODE_BLK)
    src = edge_index[0]
    dst = edge_index[1]
    pad = jnp.zeros((CH,), jnp.int32)
    dst_p = jnp.concatenate([dst, pad])
    src_p = jnp.concatenate([src, pad])
    zeros_nd = jnp.zeros((N, D), jnp.float32)
    layers = [
        (W_msg1_0, b_msg1_0, W_msg2_0, b_msg2_0, W_upd1_0, b_upd1_0, W_upd2_0, b_upd2_0),
        (W_msg1_1, b_msg1_1, W_msg2_1, b_msg2_1, W_upd1_1, b_upd1_1, W_upd2_1, b_upd2_1),
    ]
    for (Wm1, bm1, Wm2, bm2, Wu1, bu1, Wu2, bu2) in layers:
        hd, hs = _sc_gather2(h, dst_p, src_p)
        m2 = _tp2_call(hd, [(hs, 0)], edge_attr, Wm1, bm1, Wm2, bm2,
                       silu_last=True, residual=False, blk=EDGE_BLK)
        agg2 = _sc_scatter_add(m2, dst, zeros_nd)
        h = _tp2_call(h, [(agg2, 0), (agg2, N // NODE_BLK)], na,
                      Wu1, bu1, Wu2, bu2,
                      silu_last=False, residual=True, blk=NODE_BLK)
    h = _tp2_call(h, [], na, W_pre1, b_pre1, W_pre2, b_pre2,
                  silu_last=False, residual=False, blk=NODE_BLK)
    return h
